# Initial kernel scaffold; baseline (speedup 1.0000x reference)
import jax
import jax.numpy as jnp
from jax.experimental import pallas as pl


def kernel(x, gate_w1, gate_b1, gate_w2, gate_b2, up_w, up_b, down_w, down_b):
    raise NotImplementedError("write your pallas kernel here")


# speedup vs baseline: 5.4301x; 5.4301x over previous
import functools
import math

import jax
import jax.numpy as jnp
from jax import lax
from jax.experimental import pallas as pl
from jax.experimental.pallas import tpu as pltpu
from jax.experimental.pallas import tpu_sc as plsc

_D_MODEL = 768
_D_FF = 3072
_E = 16
_TEMP = 0.5
_S = 2048
_SLOTS = 2 * _S
_TILE = 128
_NTILES = _SLOTS // _TILE + _E
_ROWS = _NTILES * _TILE
_NC, _NS = 2, 16
_NW = _NC * _NS


def _router_body(x_ref, gw1_ref, gb1_ref, gw2_ref, gb2_ref,
                 probs_ref, loss_ref, wflat_ref, pos_ref, te_ref):
    x = x_ref[...]
    h = jnp.maximum(
        jnp.dot(x, gw1_ref[...], preferred_element_type=jnp.float32)
        + gb1_ref[...], 0.0)
    scores = (jnp.dot(h, gw2_ref[...], preferred_element_type=jnp.float32)
              + gb2_ref[...])
    logits = scores * (1.0 / _TEMP)
    m = jnp.max(logits, axis=-1, keepdims=True)
    ex = jnp.exp(logits - m)
    probs = ex / jnp.sum(ex, axis=-1, keepdims=True)
    probs_ref[...] = probs

    usage = jnp.mean(probs, axis=0, keepdims=True)
    loss = jnp.mean((usage - 1.0 / _E) ** 2)
    loss_ref[...] = jnp.reshape(loss, (1, 1))

    rowi = lax.broadcasted_iota(jnp.int32, (_E, _E), 0)
    coli = lax.broadcasted_iota(jnp.int32, (_E, _E), 1)
    tri_incl = (rowi <= coli).astype(jnp.float32)
    tri_strict = (rowi < coli).astype(jnp.float32)

    v1 = jnp.max(probs, axis=-1, keepdims=True)
    oh1 = (probs == v1).astype(jnp.float32)
    cs1 = jnp.dot(oh1, tri_incl, preferred_element_type=jnp.float32)
    oh1 = oh1 * (cs1 == 1.0).astype(jnp.float32)
    masked = probs - 2.0 * oh1
    v2 = jnp.max(masked, axis=-1, keepdims=True)
    oh2 = (masked == v2).astype(jnp.float32)
    cs2 = jnp.dot(oh2, tri_incl, preferred_element_type=jnp.float32)
    oh2 = oh2 * (cs2 == 1.0).astype(jnp.float32)

    e2 = jnp.exp(v2 - v1)
    w1 = 1.0 / (1.0 + e2)
    w2 = e2 * w1
    wflat_ref[0:_S, :] = w1
    wflat_ref[_S:_SLOTS, :] = w2

    counts = (jnp.sum(oh1, axis=0, keepdims=True)
              + jnp.sum(oh2, axis=0, keepdims=True))
    ptiles = jnp.floor((counts + (_TILE - 1.0)) * (1.0 / _TILE))
    seg = jnp.dot(ptiles * _TILE, tri_strict,
                  preferred_element_type=jnp.float32)

    bi = lax.broadcasted_iota(jnp.int32, (_TILE, _TILE), 0)
    bj = lax.broadcasted_iota(jnp.int32, (_TILE, _TILE), 1)
    ls = (bj < bi).astype(jnp.float32)
    pref = jnp.zeros((1, _E), dtype=jnp.float32)
    nblk = _SLOTS // _TILE
    for b in range(nblk):
        if b < nblk // 2:
            blk = oh1[b * _TILE:(b + 1) * _TILE, :]
        else:
            t0 = (b - nblk // 2) * _TILE
            blk = oh2[t0:t0 + _TILE, :]
        rank = jnp.dot(ls, blk, preferred_element_type=jnp.float32) + pref
        posb = jnp.sum((seg + rank) * blk, axis=-1, keepdims=True)
        pos_ref[b * _TILE:(b + 1) * _TILE, :] = posb.astype(jnp.int32)
        pref = pref + jnp.sum(blk, axis=0, keepdims=True)

    tbase = (lax.broadcasted_iota(jnp.int32, (_NTILES, 1), 0)
             * _TILE).astype(jnp.float32)
    te = jnp.sum((seg <= tbase).astype(jnp.int32), axis=-1, keepdims=True) - 1
    te_ref[...] = te


def _router(x2d, gw1, gb1, gw2, gb2):
    return pl.pallas_call(
        _router_body,
        out_shape=[
            jax.ShapeDtypeStruct((_S, _E), jnp.float32),
            jax.ShapeDtypeStruct((1, 1), jnp.float32),
            jax.ShapeDtypeStruct((_SLOTS, 1), jnp.float32),
            jax.ShapeDtypeStruct((_SLOTS, 1), jnp.int32),
            jax.ShapeDtypeStruct((_NTILES, 1), jnp.int32),
        ],
    )(x2d, gw1, gb1.reshape(1, -1), gw2, gb2.reshape(1, -1))


def _dispatch_body(x_hbm, pos_hbm, xs_hbm, idx_v, rows_v, s1):
    wid = lax.axis_index("s") * _NC + lax.axis_index("c")
    base = wid * _TILE
    tok = lax.rem(base, _S)
    pltpu.sync_copy(pos_hbm.at[pl.ds(base, _TILE)], idx_v)
    pltpu.sync_copy(x_hbm.at[pl.ds(tok, _TILE)], rows_v)
    pltpu.async_copy(rows_v, xs_hbm.at[idx_v], s1).wait()


def _dispatch(x2d, pos1d):
    mesh = plsc.VectorSubcoreMesh(core_axis_name="c", subcore_axis_name="s",
                                  num_cores=_NC, num_subcores=_NS)
    return pl.kernel(
        _dispatch_body,
        out_type=jax.ShapeDtypeStruct((_ROWS, _D_MODEL), jnp.float32),
        mesh=mesh,
        scratch_types=[
            pltpu.VMEM((_TILE,), jnp.int32),
            pltpu.VMEM((_TILE, _D_MODEL), jnp.float32),
            pltpu.SemaphoreType.DMA,
        ],
    )(x2d, pos1d)


def _gmm_body(te_ref, xs_ref, up_ref, ub_ref, down_ref, db_ref, out_ref):
    xb = xs_ref[...]
    h = (jnp.dot(xb, up_ref[0], preferred_element_type=jnp.float32)
         + ub_ref[0])
    g = 0.5 * h * (1.0 + lax.erf(h * (1.0 / math.sqrt(2.0))))
    o = (jnp.dot(g, down_ref[0], preferred_element_type=jnp.float32)
         + db_ref[0])
    out_ref[...] = o


def _gmm(te, xs, up_w, up_b, down_w, down_b):
    grid_spec = pltpu.PrefetchScalarGridSpec(
        num_scalar_prefetch=1,
        grid=(_NTILES,),
        in_specs=[
            pl.BlockSpec((_TILE, _D_MODEL), lambda i, te: (i, 0)),
            pl.BlockSpec((1, _D_MODEL, _D_FF), lambda i, te: (te[i], 0, 0)),
            pl.BlockSpec((1, 1, _D_FF), lambda i, te: (te[i], 0, 0)),
            pl.BlockSpec((1, _D_FF, _D_MODEL), lambda i, te: (te[i], 0, 0)),
            pl.BlockSpec((1, 1, _D_MODEL), lambda i, te: (te[i], 0, 0)),
        ],
        out_specs=pl.BlockSpec((_TILE, _D_MODEL), lambda i, te: (i, 0)),
    )
    return pl.pallas_call(
        _gmm_body,
        grid_spec=grid_spec,
        out_shape=jax.ShapeDtypeStruct((_ROWS, _D_MODEL), jnp.float32),
    )(te, xs, up_w, up_b.reshape(_E, 1, _D_FF),
      down_w, down_b.reshape(_E, 1, _D_MODEL))


def _combine_body(buf_hbm, pos_hbm, r1_hbm, r2_hbm,
                  i1_v, i2_v, a_v, b_v, s1, s2):
    wid = lax.axis_index("s") * _NC + lax.axis_index("c")
    n = _S // _NW
    base = wid * n
    pltpu.sync_copy(pos_hbm.at[pl.ds(base, n)], i1_v)
    pltpu.sync_copy(pos_hbm.at[pl.ds(_S + base, n)], i2_v)
    c1 = pltpu.async_copy(buf_hbm.at[i1_v], a_v, s1)
    c2 = pltpu.async_copy(buf_hbm.at[i2_v], b_v, s2)
    c1.wait()
    c2.wait()
    pltpu.sync_copy(a_v, r1_hbm.at[pl.ds(base, n)])
    pltpu.sync_copy(b_v, r2_hbm.at[pl.ds(base, n)])


def _combine(buf, pos1d):
    n = _S // _NW
    mesh = plsc.VectorSubcoreMesh(core_axis_name="c", subcore_axis_name="s",
                                  num_cores=_NC, num_subcores=_NS)
    return pl.kernel(
        _combine_body,
        out_type=[
            jax.ShapeDtypeStruct((_S, _D_MODEL), jnp.float32),
            jax.ShapeDtypeStruct((_S, _D_MODEL), jnp.float32),
        ],
        mesh=mesh,
        scratch_types=[
            pltpu.VMEM((n,), jnp.int32),
            pltpu.VMEM((n,), jnp.int32),
            pltpu.VMEM((n, _D_MODEL), jnp.float32),
            pltpu.VMEM((n, _D_MODEL), jnp.float32),
            pltpu.SemaphoreType.DMA,
            pltpu.SemaphoreType.DMA,
        ],
    )(buf, pos1d)


def _wadd_body(a_ref, b_ref, wa_ref, wb_ref, o_ref):
    o_ref[...] = a_ref[...] * wa_ref[...] + b_ref[...] * wb_ref[...]


def _wadd(a, b, wa, wb):
    spec = pl.BlockSpec((_TILE, _D_MODEL), lambda i: (i, 0))
    wspec = pl.BlockSpec((_TILE, 1), lambda i: (i, 0))
    return pl.pallas_call(
        _wadd_body,
        grid=(_S // _TILE,),
        in_specs=[spec, spec, wspec, wspec],
        out_specs=spec,
        out_shape=jax.ShapeDtypeStruct((_S, _D_MODEL), jnp.float32),
    )(a, b, wa, wb)


def kernel(x, gate_w1, gate_b1, gate_w2, gate_b2, up_w, up_b, down_w, down_b):
    B, S, D = x.shape
    x2d = x.reshape(S, D)
    probs, loss, wflat, pos, te = _router(x2d, gate_w1, gate_b1,
                                          gate_w2, gate_b2)
    pos1d = pos.reshape(_SLOTS)
    xs = _dispatch(x2d, pos1d)
    buf = _gmm(te.reshape(_NTILES), xs, up_w, up_b, down_w, down_b)
    r1, r2 = _combine(buf, pos1d)
    out = _wadd(r1, r2, wflat[:_S], wflat[_S:])
    return (out.reshape(B, S, D), loss.reshape(()), probs.reshape(B, S, _E))


# speedup vs baseline: 5.6503x; 1.0406x over previous
import functools
import math

import jax
import jax.numpy as jnp
from jax import lax
from jax.experimental import pallas as pl
from jax.experimental.pallas import tpu as pltpu
from jax.experimental.pallas import tpu_sc as plsc

_D_MODEL = 768
_D_FF = 3072
_E = 16
_TEMP = 0.5
_S = 2048
_SLOTS = 2 * _S
_TILE = 128
_NTILES = _SLOTS // _TILE + _E
_ROWS = _NTILES * _TILE
_NC, _NS = 2, 16
_NW = _NC * _NS


def _router_body(x_ref, gw1_ref, gb1_ref, gw2_ref, gb2_ref,
                 probs_ref, loss_ref, wflat_ref, pos_ref, te_ref):
    x = x_ref[...]
    h = jnp.maximum(
        jnp.dot(x, gw1_ref[...], preferred_element_type=jnp.float32)
        + gb1_ref[...], 0.0)
    scores = (jnp.dot(h, gw2_ref[...], preferred_element_type=jnp.float32)
              + gb2_ref[...])
    logits = scores * (1.0 / _TEMP)
    m = jnp.max(logits, axis=-1, keepdims=True)
    ex = jnp.exp(logits - m)
    probs = ex / jnp.sum(ex, axis=-1, keepdims=True)
    probs_ref[...] = probs

    usage = jnp.mean(probs, axis=0, keepdims=True)
    loss = jnp.mean((usage - 1.0 / _E) ** 2)
    loss_ref[...] = jnp.reshape(loss, (1, 1))

    rowi = lax.broadcasted_iota(jnp.int32, (_E, _E), 0)
    coli = lax.broadcasted_iota(jnp.int32, (_E, _E), 1)
    tri_incl = (rowi <= coli).astype(jnp.float32)
    tri_strict = (rowi < coli).astype(jnp.float32)

    v1 = jnp.max(probs, axis=-1, keepdims=True)
    oh1 = (probs == v1).astype(jnp.float32)
    cs1 = jnp.dot(oh1, tri_incl, preferred_element_type=jnp.float32)
    oh1 = oh1 * (cs1 == 1.0).astype(jnp.float32)
    masked = probs - 2.0 * oh1
    v2 = jnp.max(masked, axis=-1, keepdims=True)
    oh2 = (masked == v2).astype(jnp.float32)
    cs2 = jnp.dot(oh2, tri_incl, preferred_element_type=jnp.float32)
    oh2 = oh2 * (cs2 == 1.0).astype(jnp.float32)

    e2 = jnp.exp(v2 - v1)
    w1 = 1.0 / (1.0 + e2)
    w2 = e2 * w1
    wflat_ref[0:_S, :] = w1
    wflat_ref[_S:_SLOTS, :] = w2

    counts = (jnp.sum(oh1, axis=0, keepdims=True)
              + jnp.sum(oh2, axis=0, keepdims=True))
    ptiles = jnp.floor((counts + (_TILE - 1.0)) * (1.0 / _TILE))
    seg = jnp.dot(ptiles * _TILE, tri_strict,
                  preferred_element_type=jnp.float32)

    bi = lax.broadcasted_iota(jnp.int32, (_TILE, _TILE), 0)
    bj = lax.broadcasted_iota(jnp.int32, (_TILE, _TILE), 1)
    ls = (bj < bi).astype(jnp.float32)
    pref = jnp.zeros((1, _E), dtype=jnp.float32)
    nblk = _SLOTS // _TILE
    for b in range(nblk):
        if b < nblk // 2:
            blk = oh1[b * _TILE:(b + 1) * _TILE, :]
        else:
            t0 = (b - nblk // 2) * _TILE
            blk = oh2[t0:t0 + _TILE, :]
        rank = jnp.dot(ls, blk, preferred_element_type=jnp.float32) + pref
        posb = jnp.sum((seg + rank) * blk, axis=-1, keepdims=True)
        pos_ref[b * _TILE:(b + 1) * _TILE, :] = posb.astype(jnp.int32)
        pref = pref + jnp.sum(blk, axis=0, keepdims=True)

    tbase = (lax.broadcasted_iota(jnp.int32, (_NTILES, 1), 0)
             * _TILE).astype(jnp.float32)
    te = jnp.sum((seg <= tbase).astype(jnp.int32), axis=-1, keepdims=True) - 1
    te_ref[0:_NTILES, :] = te
    nt = jnp.sum(ptiles, axis=-1, keepdims=True).astype(jnp.int32)
    te_ref[_NTILES:_NTILES + 1, :] = nt


def _router(x2d, gw1, gb1, gw2, gb2):
    return pl.pallas_call(
        _router_body,
        out_shape=[
            jax.ShapeDtypeStruct((_S, _E), jnp.float32),
            jax.ShapeDtypeStruct((1, 1), jnp.float32),
            jax.ShapeDtypeStruct((_SLOTS, 1), jnp.float32),
            jax.ShapeDtypeStruct((_SLOTS, 1), jnp.int32),
            jax.ShapeDtypeStruct((_NTILES + 1, 1), jnp.int32),
        ],
    )(x2d, gw1, gb1.reshape(1, -1), gw2, gb2.reshape(1, -1))


def _dispatch_body(x_hbm, pos_hbm, xs_hbm, idx_v, rows_v, s1):
    wid = lax.axis_index("s") * _NC + lax.axis_index("c")
    base = wid * _TILE
    tok = lax.rem(base, _S)
    pltpu.sync_copy(pos_hbm.at[pl.ds(base, _TILE)], idx_v)
    pltpu.sync_copy(x_hbm.at[pl.ds(tok, _TILE)], rows_v)
    pltpu.async_copy(rows_v, xs_hbm.at[idx_v], s1).wait()


def _dispatch(x2d, pos1d):
    mesh = plsc.VectorSubcoreMesh(core_axis_name="c", subcore_axis_name="s",
                                  num_cores=_NC, num_subcores=_NS)
    return pl.kernel(
        _dispatch_body,
        out_type=jax.ShapeDtypeStruct((_ROWS, _D_MODEL), jnp.float32),
        mesh=mesh,
        scratch_types=[
            pltpu.VMEM((_TILE,), jnp.int32),
            pltpu.VMEM((_TILE, _D_MODEL), jnp.float32),
            pltpu.SemaphoreType.DMA,
        ],
    )(x2d, pos1d)


def _gmm_body(te_ref, xs_ref, up_ref, ub_ref, down_ref, db_ref, out_ref):
    @pl.when(pl.program_id(0) < te_ref[_NTILES])
    def _():
        xb = xs_ref[...]
        h = (jnp.dot(xb, up_ref[0], preferred_element_type=jnp.float32)
             + ub_ref[0])
        g = 0.5 * h * (1.0 + lax.erf(h * (1.0 / math.sqrt(2.0))))
        o = (jnp.dot(g, down_ref[0], preferred_element_type=jnp.float32)
             + db_ref[0])
        out_ref[...] = o


def _gmm(te, xs, up_w, up_b, down_w, down_b):
    grid_spec = pltpu.PrefetchScalarGridSpec(
        num_scalar_prefetch=1,
        grid=(_NTILES,),
        in_specs=[
            pl.BlockSpec((_TILE, _D_MODEL), lambda i, te: (i, 0)),
            pl.BlockSpec((1, _D_MODEL, _D_FF), lambda i, te: (te[i], 0, 0)),
            pl.BlockSpec((1, 1, _D_FF), lambda i, te: (te[i], 0, 0)),
            pl.BlockSpec((1, _D_FF, _D_MODEL), lambda i, te: (te[i], 0, 0)),
            pl.BlockSpec((1, 1, _D_MODEL), lambda i, te: (te[i], 0, 0)),
        ],
        out_specs=pl.BlockSpec((_TILE, _D_MODEL), lambda i, te: (i, 0)),
    )
    return pl.pallas_call(
        _gmm_body,
        grid_spec=grid_spec,
        out_shape=jax.ShapeDtypeStruct((_ROWS, _D_MODEL), jnp.float32),
    )(te, xs, up_w, up_b.reshape(_E, 1, _D_FF),
      down_w, down_b.reshape(_E, 1, _D_MODEL))


def _combine_body(buf_hbm, pos_hbm, r1_hbm, r2_hbm,
                  i1_v, i2_v, a_v, b_v, s1, s2):
    wid = lax.axis_index("s") * _NC + lax.axis_index("c")
    n = _S // _NW
    base = wid * n
    pltpu.sync_copy(pos_hbm.at[pl.ds(base, n)], i1_v)
    pltpu.sync_copy(pos_hbm.at[pl.ds(_S + base, n)], i2_v)
    c1 = pltpu.async_copy(buf_hbm.at[i1_v], a_v, s1)
    c2 = pltpu.async_copy(buf_hbm.at[i2_v], b_v, s2)
    c1.wait()
    c2.wait()
    pltpu.sync_copy(a_v, r1_hbm.at[pl.ds(base, n)])
    pltpu.sync_copy(b_v, r2_hbm.at[pl.ds(base, n)])


def _combine(buf, pos1d):
    n = _S // _NW
    mesh = plsc.VectorSubcoreMesh(core_axis_name="c", subcore_axis_name="s",
                                  num_cores=_NC, num_subcores=_NS)
    return pl.kernel(
        _combine_body,
        out_type=[
            jax.ShapeDtypeStruct((_S, _D_MODEL), jnp.float32),
            jax.ShapeDtypeStruct((_S, _D_MODEL), jnp.float32),
        ],
        mesh=mesh,
        scratch_types=[
            pltpu.VMEM((n,), jnp.int32),
            pltpu.VMEM((n,), jnp.int32),
            pltpu.VMEM((n, _D_MODEL), jnp.float32),
            pltpu.VMEM((n, _D_MODEL), jnp.float32),
            pltpu.SemaphoreType.DMA,
            pltpu.SemaphoreType.DMA,
        ],
    )(buf, pos1d)


def _wadd_body(a_ref, b_ref, wa_ref, wb_ref, o_ref):
    o_ref[...] = a_ref[...] * wa_ref[...] + b_ref[...] * wb_ref[...]


def _wadd(a, b, wa, wb):
    spec = pl.BlockSpec((_TILE, _D_MODEL), lambda i: (i, 0))
    wspec = pl.BlockSpec((_TILE, 1), lambda i: (i, 0))
    return pl.pallas_call(
        _wadd_body,
        grid=(_S // _TILE,),
        in_specs=[spec, spec, wspec, wspec],
        out_specs=spec,
        out_shape=jax.ShapeDtypeStruct((_S, _D_MODEL), jnp.float32),
    )(a, b, wa, wb)


def kernel(x, gate_w1, gate_b1, gate_w2, gate_b2, up_w, up_b, down_w, down_b):
    B, S, D = x.shape
    x2d = x.reshape(S, D)
    probs, loss, wflat, pos, te = _router(x2d, gate_w1, gate_b1,
                                          gate_w2, gate_b2)
    pos1d = pos.reshape(_SLOTS)
    xs = _dispatch(x2d, pos1d)
    buf = _gmm(te.reshape(_NTILES + 1), xs, up_w, up_b, down_w, down_b)
    r1, r2 = _combine(buf, pos1d)
    out = _wadd(r1, r2, wflat[:_S], wflat[_S:])
    return (out.reshape(B, S, D), loss.reshape(()), probs.reshape(B, S, _E))


# speedup vs baseline: 6.1470x; 1.0879x over previous
import functools
import math

import jax
import jax.numpy as jnp
from jax import lax
from jax.experimental import pallas as pl
from jax.experimental.pallas import tpu as pltpu
from jax.experimental.pallas import tpu_sc as plsc

_D_MODEL = 768
_D_FF = 3072
_E = 16
_TEMP = 0.5
_S = 2048
_SLOTS = 2 * _S
_TILE = 128
_NTILES = _SLOTS // _TILE + _E
_ROWS = _NTILES * _TILE
_NC, _NS = 2, 16
_NW = _NC * _NS


def _router_body(x_ref, gw1_ref, gb1_ref, gw2_ref, gb2_ref,
                 probs_ref, loss_ref, wflat_ref, pos_ref, te_ref):
    x = x_ref[...]
    h = jnp.maximum(
        jnp.dot(x, gw1_ref[...], preferred_element_type=jnp.float32)
        + gb1_ref[...], 0.0)
    scores = (jnp.dot(h, gw2_ref[...], preferred_element_type=jnp.float32)
              + gb2_ref[...])
    logits = scores * (1.0 / _TEMP)
    m = jnp.max(logits, axis=-1, keepdims=True)
    ex = jnp.exp(logits - m)
    probs = ex / jnp.sum(ex, axis=-1, keepdims=True)
    probs_ref[...] = probs

    usage = jnp.mean(probs, axis=0, keepdims=True)
    loss = jnp.mean((usage - 1.0 / _E) ** 2)
    loss_ref[...] = jnp.reshape(loss, (1, 1))

    rowi = lax.broadcasted_iota(jnp.int32, (_E, _E), 0)
    coli = lax.broadcasted_iota(jnp.int32, (_E, _E), 1)
    tri_incl = (rowi <= coli).astype(jnp.float32)
    tri_strict = (rowi < coli).astype(jnp.float32)

    v1 = jnp.max(probs, axis=-1, keepdims=True)
    oh1 = (probs == v1).astype(jnp.float32)
    cs1 = jnp.dot(oh1, tri_incl, preferred_element_type=jnp.float32)
    oh1 = oh1 * (cs1 == 1.0).astype(jnp.float32)
    masked = probs - 2.0 * oh1
    v2 = jnp.max(masked, axis=-1, keepdims=True)
    oh2 = (masked == v2).astype(jnp.float32)
    cs2 = jnp.dot(oh2, tri_incl, preferred_element_type=jnp.float32)
    oh2 = oh2 * (cs2 == 1.0).astype(jnp.float32)

    e2 = jnp.exp(v2 - v1)
    w1 = 1.0 / (1.0 + e2)
    w2 = e2 * w1
    wflat_ref[0:_S, :] = w1
    wflat_ref[_S:_SLOTS, :] = w2

    counts = (jnp.sum(oh1, axis=0, keepdims=True)
              + jnp.sum(oh2, axis=0, keepdims=True))
    ptiles = jnp.floor((counts + (_TILE - 1.0)) * (1.0 / _TILE))
    seg = jnp.dot(ptiles * _TILE, tri_strict,
                  preferred_element_type=jnp.float32)

    bi = lax.broadcasted_iota(jnp.int32, (_TILE, _TILE), 0)
    bj = lax.broadcasted_iota(jnp.int32, (_TILE, _TILE), 1)
    ls = (bj < bi).astype(jnp.float32)
    pref = jnp.zeros((1, _E), dtype=jnp.float32)
    nblk = _SLOTS // _TILE
    for b in range(nblk):
        if b < nblk // 2:
            blk = oh1[b * _TILE:(b + 1) * _TILE, :]
        else:
            t0 = (b - nblk // 2) * _TILE
            blk = oh2[t0:t0 + _TILE, :]
        rank = jnp.dot(ls, blk, preferred_element_type=jnp.float32) + pref
        posb = jnp.sum((seg + rank) * blk, axis=-1, keepdims=True)
        pos_ref[b * _TILE:(b + 1) * _TILE, :] = posb.astype(jnp.int32)
        pref = pref + jnp.sum(blk, axis=0, keepdims=True)

    tbase = (lax.broadcasted_iota(jnp.int32, (_NTILES, 1), 0)
             * _TILE).astype(jnp.float32)
    te = jnp.sum((seg <= tbase).astype(jnp.int32), axis=-1, keepdims=True) - 1
    te_ref[0:_NTILES, :] = te
    nt = jnp.sum(ptiles, axis=-1, keepdims=True).astype(jnp.int32)
    te_ref[_NTILES:_NTILES + 1, :] = nt


def _router(x2d, gw1, gb1, gw2, gb2):
    return pl.pallas_call(
        _router_body,
        out_shape=[
            jax.ShapeDtypeStruct((_S, _E), jnp.float32),
            jax.ShapeDtypeStruct((1, 1), jnp.float32),
            jax.ShapeDtypeStruct((_SLOTS, 1), jnp.float32),
            jax.ShapeDtypeStruct((_SLOTS, 1), jnp.int32),
            jax.ShapeDtypeStruct((_NTILES + 1, 1), jnp.int32),
        ],
    )(x2d, gw1, gb1.reshape(1, -1), gw2, gb2.reshape(1, -1))


def _dispatch_body(x_hbm, pos_hbm, xs_hbm, idx0, idx1, rows0, rows1,
                   ls0, ls1, ss0, ss1):
    wid = lax.axis_index("s") * _NC + lax.axis_index("c")
    base = wid * _TILE
    tok = lax.rem(base, _S)
    hh = _TILE // 2
    pltpu.sync_copy(pos_hbm.at[pl.ds(base, hh)], idx0)
    pltpu.sync_copy(pos_hbm.at[pl.ds(base + hh, hh)], idx1)
    l0 = pltpu.make_async_copy(x_hbm.at[pl.ds(tok, hh)], rows0, ls0)
    l0.start()
    l1 = pltpu.make_async_copy(x_hbm.at[pl.ds(tok + hh, hh)], rows1, ls1)
    l1.start()
    l0.wait()
    s0 = pltpu.async_copy(rows0, xs_hbm.at[idx0], ss0)
    l1.wait()
    s1 = pltpu.async_copy(rows1, xs_hbm.at[idx1], ss1)
    s0.wait()
    s1.wait()


def _dispatch(x2d, pos1d):
    hh = _TILE // 2
    mesh = plsc.VectorSubcoreMesh(core_axis_name="c", subcore_axis_name="s",
                                  num_cores=_NC, num_subcores=_NS)
    return pl.kernel(
        _dispatch_body,
        out_type=jax.ShapeDtypeStruct((_ROWS, _D_MODEL), jnp.float32),
        mesh=mesh,
        scratch_types=[
            pltpu.VMEM((hh,), jnp.int32),
            pltpu.VMEM((hh,), jnp.int32),
            pltpu.VMEM((hh, _D_MODEL), jnp.float32),
            pltpu.VMEM((hh, _D_MODEL), jnp.float32),
            pltpu.SemaphoreType.DMA,
            pltpu.SemaphoreType.DMA,
            pltpu.SemaphoreType.DMA,
            pltpu.SemaphoreType.DMA,
        ],
    )(x2d, pos1d)


def _gmm_body(te_ref, sched_ref, xs_ref, up_hbm, ub_ref, down_hbm, db_ref,
              out_ref, up_buf, down_buf, sems):
    i = pl.program_id(0)
    first = sched_ref[i, 0]
    par = sched_ref[i, 1]
    nexp = sched_ref[i, 2]
    hasnext = sched_ref[i, 3]

    @pl.when(i == 0)
    def _():
        e0 = te_ref[0]
        pltpu.make_async_copy(up_hbm.at[e0], up_buf.at[0],
                              sems.at[0, 0]).start()
        pltpu.make_async_copy(down_hbm.at[e0], down_buf.at[0],
                              sems.at[0, 1]).start()

    @pl.when((first == 1) & (hasnext == 1))
    def _():
        ns = 1 - par
        pltpu.make_async_copy(up_hbm.at[nexp], up_buf.at[ns],
                              sems.at[ns, 0]).start()
        pltpu.make_async_copy(down_hbm.at[nexp], down_buf.at[ns],
                              sems.at[ns, 1]).start()

    @pl.when(first == 1)
    def _():
        e = te_ref[i]
        pltpu.make_async_copy(up_hbm.at[e], up_buf.at[par],
                              sems.at[par, 0]).wait()
        pltpu.make_async_copy(down_hbm.at[e], down_buf.at[par],
                              sems.at[par, 1]).wait()

    @pl.when(i < te_ref[_NTILES])
    def _():
        xb = xs_ref[...]
        h = (jnp.dot(xb, up_buf[par], preferred_element_type=jnp.float32)
             + ub_ref[0])
        g = 0.5 * h * (1.0 + lax.erf(h * (1.0 / math.sqrt(2.0))))
        o = (jnp.dot(g, down_buf[par], preferred_element_type=jnp.float32)
             + db_ref[0])
        out_ref[...] = o


def _gmm(te, sched, xs, up_w, up_b, down_w, down_b):
    grid_spec = pltpu.PrefetchScalarGridSpec(
        num_scalar_prefetch=2,
        grid=(_NTILES,),
        in_specs=[
            pl.BlockSpec((_TILE, _D_MODEL), lambda i, te, sc: (i, 0)),
            pl.BlockSpec(memory_space=pl.ANY),
            pl.BlockSpec((1, 1, _D_FF), lambda i, te, sc: (te[i], 0, 0)),
            pl.BlockSpec(memory_space=pl.ANY),
            pl.BlockSpec((1, 1, _D_MODEL), lambda i, te, sc: (te[i], 0, 0)),
        ],
        out_specs=pl.BlockSpec(
            (_TILE, _D_MODEL),
            lambda i, te, sc: (jnp.where(i < te[_NTILES], i, _NTILES), 0)),
        scratch_shapes=[
            pltpu.VMEM((2, _D_MODEL, _D_FF), jnp.float32),
            pltpu.VMEM((2, _D_FF, _D_MODEL), jnp.float32),
            pltpu.SemaphoreType.DMA((2, 2)),
        ],
    )
    return pl.pallas_call(
        _gmm_body,
        grid_spec=grid_spec,
        out_shape=jax.ShapeDtypeStruct((_ROWS + _TILE, _D_MODEL), jnp.float32),
    )(te, sched, xs, up_w, up_b.reshape(_E, 1, _D_FF),
      down_w, down_b.reshape(_E, 1, _D_MODEL))


def _combine_body(buf_hbm, pos_hbm, r1_hbm, r2_hbm,
                  i1_v, i2_v, a_v, b_v, s1, s2):
    wid = lax.axis_index("s") * _NC + lax.axis_index("c")
    n = _S // _NW
    base = wid * n
    pltpu.sync_copy(pos_hbm.at[pl.ds(base, n)], i1_v)
    pltpu.sync_copy(pos_hbm.at[pl.ds(_S + base, n)], i2_v)
    c1 = pltpu.async_copy(buf_hbm.at[i1_v], a_v, s1)
    c2 = pltpu.async_copy(buf_hbm.at[i2_v], b_v, s2)
    c1.wait()
    w1 = pltpu.make_async_copy(a_v, r1_hbm.at[pl.ds(base, n)], s1)
    w1.start()
    c2.wait()
    w2 = pltpu.make_async_copy(b_v, r2_hbm.at[pl.ds(base, n)], s2)
    w2.start()
    w1.wait()
    w2.wait()


def _combine(buf, pos1d):
    n = _S // _NW
    mesh = plsc.VectorSubcoreMesh(core_axis_name="c", subcore_axis_name="s",
                                  num_cores=_NC, num_subcores=_NS)
    return pl.kernel(
        _combine_body,
        out_type=[
            jax.ShapeDtypeStruct((_S, _D_MODEL), jnp.float32),
            jax.ShapeDtypeStruct((_S, _D_MODEL), jnp.float32),
        ],
        mesh=mesh,
        scratch_types=[
            pltpu.VMEM((n,), jnp.int32),
            pltpu.VMEM((n,), jnp.int32),
            pltpu.VMEM((n, _D_MODEL), jnp.float32),
            pltpu.VMEM((n, _D_MODEL), jnp.float32),
            pltpu.SemaphoreType.DMA,
            pltpu.SemaphoreType.DMA,
        ],
    )(buf, pos1d)


def _wadd_body(a_ref, b_ref, wa_ref, wb_ref, o_ref):
    o_ref[...] = a_ref[...] * wa_ref[...] + b_ref[...] * wb_ref[...]


def _wadd(a, b, wa, wb):
    spec = pl.BlockSpec((_TILE, _D_MODEL), lambda i: (i, 0))
    wspec = pl.BlockSpec((_TILE, 1), lambda i: (i, 0))
    return pl.pallas_call(
        _wadd_body,
        grid=(_S // _TILE,),
        in_specs=[spec, spec, wspec, wspec],
        out_specs=spec,
        out_shape=jax.ShapeDtypeStruct((_S, _D_MODEL), jnp.float32),
    )(a, b, wa, wb)


def kernel(x, gate_w1, gate_b1, gate_w2, gate_b2, up_w, up_b, down_w, down_b):
    B, S, D = x.shape
    x2d = x.reshape(S, D)
    probs, loss, wflat, pos, te = _router(x2d, gate_w1, gate_b1,
                                          gate_w2, gate_b2)
    pos1d = pos.reshape(_SLOTS)
    xs = _dispatch(x2d, pos1d)

    te48 = te.reshape(_NTILES + 1)[:_NTILES]
    nt = te.reshape(_NTILES + 1)[_NTILES]
    steps = jnp.arange(_NTILES, dtype=jnp.int32)
    valid = steps < nt
    first = valid & jnp.concatenate(
        [jnp.ones((1,), bool), te48[1:] != te48[:-1]])
    kk = jnp.cumsum(first.astype(jnp.int32)) - 1
    par = kk % 2
    ranks = jnp.arange(_E + 1, dtype=jnp.int32)
    oh48 = (kk[None, :] == ranks[:, None]) & first[None, :]
    seqexp = jnp.sum(oh48 * te48[None, :], axis=1)
    nseq = jnp.sum(first.astype(jnp.int32))
    nexpert = seqexp[jnp.clip(kk + 1, 0, _E)]
    hasnext = ((kk + 1) < nseq) & valid
    sched = jnp.stack([first.astype(jnp.int32), par, nexpert,
                       hasnext.astype(jnp.int32)], axis=1)

    buf = _gmm(te.reshape(_NTILES + 1), sched, xs, up_w, up_b, down_w, down_b)
    r1, r2 = _combine(buf, pos1d)
    out = _wadd(r1, r2, wflat[:_S], wflat[_S:])
    return (out.reshape(B, S, D), loss.reshape(()), probs.reshape(B, S, _E))
